# baseline (speedup 1.0000x reference)
import functools

import jax
import jax.numpy as jnp
from jax import lax
from jax.experimental import pallas as pl
from jax.experimental.pallas import tpu as pltpu
from jax.experimental.pallas import tpu_sc as plsc

_NC = 2
_NS = 16
_L = 16
_NW = _NC * _NS

_ROWS = 40


def _sc_lookup(conv_tensor, sp_t):
    nrows, ncols = sp_t.shape
    stripe = ncols // _NW
    n_chunks = nrows // _ROWS
    nvec = stripe // _L
    mesh = plsc.VectorSubcoreMesh(core_axis_name="c", subcore_axis_name="s")

    @functools.partial(
        pl.kernel,
        out_type=jax.ShapeDtypeStruct((nrows, ncols), jnp.int32),
        mesh=mesh,
        scratch_types=(
            [pltpu.VMEM((conv_tensor.shape[0],), jnp.int32)]
            + [pltpu.VMEM((_ROWS, stripe), jnp.int32)] * 6
            + [pltpu.SemaphoreType.DMA] * 6
        ),
        compiler_params=pltpu.CompilerParams(
            needs_layout_passes=False, use_tc_tiling_on_sc=True),
    )
    def k(conv_hbm, sp_hbm, out_hbm, table_v, *rest):
        ins, outs = rest[0:3], rest[3:6]
        isems, osems = rest[6:9], rest[9:12]
        wid = lax.axis_index("s") * _NC + lax.axis_index("c")
        col0 = wid * stripe
        pltpu.sync_copy(conv_hbm, table_v)

        def src(c):
            return sp_hbm.at[pl.ds(c * _ROWS, _ROWS), pl.ds(col0, stripe)]

        def dst(c):
            return out_hbm.at[pl.ds(c * _ROWS, _ROWS), pl.ds(col0, stripe)]

        for b in range(3):
            pltpu.async_copy(src(b), ins[b], isems[b])

        for c in range(n_chunks):
            b = c % 3
            pltpu.make_async_copy(src(c), ins[b], isems[b]).wait()
            if c >= 3:
                pltpu.make_async_copy(outs[b], dst(c), osems[b]).wait()

            @plsc.parallel_loop(0, _ROWS, unroll=2)
            def _row(r, _in=ins[b], _out=outs[b]):
                for j in range(nvec):
                    s = pl.ds(j * _L, _L)
                    _out[r, s] = plsc.load_gather(table_v, [_in[r, s]])

            pltpu.async_copy(outs[b], dst(c), osems[b])
            if c + 3 < n_chunks:
                pltpu.async_copy(src(c + 3), ins[b], isems[b])

        for c in range(max(0, n_chunks - 3), n_chunks):
            pltpu.make_async_copy(outs[c % 3], dst(c), osems[c % 3]).wait()

    return k(conv_tensor, sp_t)


def kernel(species, conv_tensor):
    return _sc_lookup(conv_tensor, species.T).T


# speedup vs baseline: 1.0080x; 1.0080x over previous
import functools

import jax
import jax.numpy as jnp
from jax import lax
from jax.experimental import pallas as pl
from jax.experimental.pallas import tpu as pltpu
from jax.experimental.pallas import tpu_sc as plsc

_NC = 2
_NS = 16
_L = 16
_NW = _NC * _NS

_ROWS = 40


def _sc_lookup(conv_tensor, sp_t):
    nrows, ncols = sp_t.shape
    stripe = ncols // _NW
    n_chunks = nrows // _ROWS
    nvec = stripe // _L
    mesh = plsc.VectorSubcoreMesh(core_axis_name="c", subcore_axis_name="s")

    @functools.partial(
        pl.kernel,
        out_type=jax.ShapeDtypeStruct((nrows, ncols), jnp.int32),
        mesh=mesh,
        scratch_types=[
            pltpu.VMEM((conv_tensor.shape[0],), jnp.int32),
            pltpu.VMEM((_ROWS, stripe), jnp.int32),
            pltpu.VMEM((_ROWS, stripe), jnp.int32),
            pltpu.VMEM((_ROWS, stripe), jnp.int32),
            pltpu.VMEM((_ROWS, stripe), jnp.int32),
            pltpu.SemaphoreType.DMA,
            pltpu.SemaphoreType.DMA,
            pltpu.SemaphoreType.DMA,
            pltpu.SemaphoreType.DMA,
        ],
        compiler_params=pltpu.CompilerParams(
            needs_layout_passes=False, use_tc_tiling_on_sc=True),
    )
    def k(conv_hbm, sp_hbm, out_hbm, table_v, in0, in1, out0, out1,
          si0, si1, so0, so1):
        ins, outs = (in0, in1), (out0, out1)
        isems, osems = (si0, si1), (so0, so1)
        wid = lax.axis_index("s") * _NC + lax.axis_index("c")
        col0 = wid * stripe
        pltpu.sync_copy(conv_hbm, table_v)

        def src(c):
            return sp_hbm.at[pl.ds(c * _ROWS, _ROWS), pl.ds(col0, stripe)]

        def dst(c):
            return out_hbm.at[pl.ds(c * _ROWS, _ROWS), pl.ds(col0, stripe)]

        for b in range(2):
            pltpu.async_copy(src(b), ins[b], isems[b])

        for c in range(n_chunks):
            b = c % 2
            pltpu.make_async_copy(src(c), ins[b], isems[b]).wait()
            if c >= 2:
                pltpu.make_async_copy(outs[b], dst(c), osems[b]).wait()

            @plsc.parallel_loop(0, _ROWS, unroll=2)
            def _row(r, _in=ins[b], _out=outs[b]):
                for j in range(nvec):
                    s = pl.ds(j * _L, _L)
                    _out[r, s] = plsc.load_gather(table_v, [_in[r, s]])

            pltpu.async_copy(outs[b], dst(c), osems[b])
            if c + 2 < n_chunks:
                pltpu.async_copy(src(c + 2), ins[b], isems[b])

        for b in range(min(2, n_chunks)):
            pltpu.make_async_copy(outs[b], dst(0), osems[b]).wait()

    return k(conv_tensor, sp_t)


def kernel(species, conv_tensor):
    return _sc_lookup(conv_tensor, species.T).T


# speedup vs baseline: 1.0265x; 1.0183x over previous
import functools

import jax
import jax.numpy as jnp
from jax import lax
from jax.experimental import pallas as pl
from jax.experimental.pallas import tpu as pltpu
from jax.experimental.pallas import tpu_sc as plsc

_NC = 2
_NS = 16
_L = 16
_NW = _NC * _NS

_ROWS = 40


def _sc_lookup(conv_tensor, sp_t):
    nrows, ncols = sp_t.shape
    stripe = ncols // _NW
    n_chunks = nrows // _ROWS
    nvec = stripe // _L
    mesh = plsc.VectorSubcoreMesh(core_axis_name="c", subcore_axis_name="s")

    @functools.partial(
        pl.kernel,
        out_type=jax.ShapeDtypeStruct((nrows, ncols), jnp.int32),
        mesh=mesh,
        scratch_types=[
            pltpu.VMEM((conv_tensor.shape[0],), jnp.int32),
            pltpu.VMEM((_ROWS, stripe), jnp.int32),
            pltpu.VMEM((_ROWS, stripe), jnp.int32),
            pltpu.VMEM((_ROWS, stripe), jnp.int32),
            pltpu.VMEM((_ROWS, stripe), jnp.int32),
            pltpu.SemaphoreType.DMA,
            pltpu.SemaphoreType.DMA,
            pltpu.SemaphoreType.DMA,
            pltpu.SemaphoreType.DMA,
        ],
        compiler_params=pltpu.CompilerParams(
            needs_layout_passes=False, use_tc_tiling_on_sc=True),
    )
    def k(conv_hbm, sp_hbm, out_hbm, table_v, in0, in1, out0, out1,
          si0, si1, so0, so1):
        ins, outs = (in0, in1), (out0, out1)
        isems, osems = (si0, si1), (so0, so1)
        wid = lax.axis_index("s") * _NC + lax.axis_index("c")
        col0 = wid * stripe

        def src(c):
            return sp_hbm.at[pl.ds(c * _ROWS, _ROWS), pl.ds(col0, stripe)]

        def dst(c):
            return out_hbm.at[pl.ds(c * _ROWS, _ROWS), pl.ds(col0, stripe)]

        for b in range(2):
            pltpu.async_copy(src(b), ins[b], isems[b])
        pltpu.sync_copy(conv_hbm, table_v)

        for c in range(n_chunks):
            b = c % 2
            pltpu.make_async_copy(src(c), ins[b], isems[b]).wait()
            if c >= 2:
                pltpu.make_async_copy(outs[b], dst(c), osems[b]).wait()

            @plsc.parallel_loop(0, _ROWS, unroll=2)
            def _row(r, _in=ins[b], _out=outs[b]):
                for j in range(nvec):
                    s = pl.ds(j * _L, _L)
                    _out[r, s] = plsc.load_gather(table_v, [_in[r, s]])

            pltpu.async_copy(outs[b], dst(c), osems[b])
            if c + 2 < n_chunks:
                pltpu.async_copy(src(c + 2), ins[b], isems[b])

        for b in range(min(2, n_chunks)):
            pltpu.make_async_copy(outs[b], dst(0), osems[b]).wait()

    return k(conv_tensor, sp_t)


def kernel(species, conv_tensor):
    return _sc_lookup(conv_tensor, species.T).T
